# baseline (speedup 1.0000x reference)
import functools

import jax
import jax.numpy as jnp
from jax import lax
from jax.experimental import pallas as pl
from jax.experimental.pallas import tpu as pltpu
from jax.experimental.pallas import tpu_sc as plsc

_NEU_IN = 100000
_NEU_OUT = 128
_THRES = 1.0
_DECAY = 2.0 ** 4

_NW = 32
_ROWS_W = 3200
_PAD_IN = _NW * _ROWS_W
_CHUNKS = _ROWS_W // 16
_GR = 160
_NB = 5
_V8 = _NEU_OUT // 16


def _sc_body(spikes_hbm, w_hbm, out_hbm, spk_v, spkf_v,
             b0, b1, b2, b3, b4, acc_v, s0, s1, s2, s3, s4):
    bufs = (b0, b1, b2, b3, b4)
    sems = (s0, s1, s2, s3, s4)

    wid = lax.axis_index("s") * 2 + lax.axis_index("c")
    base = wid * _ROWS_W
    nvalid = jnp.minimum(_ROWS_W, _NEU_IN - base)
    trips = lax.div(nvalid, _GR)
    revs = lax.div(trips, _NB)

    pltpu.sync_copy(spikes_hbm.at[pl.ds(base, _ROWS_W)], spk_v)

    def conv(c, x):
        spkf_v[pl.ds(c * 16, 16)] = spk_v[pl.ds(c * 16, 16)].astype(jnp.float32)
        return x

    lax.fori_loop(0, _CHUNKS, conv, 0)

    def fire(c, buf, sem):
        pltpu.async_copy(w_hbm.at[pl.ds(base + c * _GR, _GR)], buf, sem)

    def drain(buf, sem):
        pltpu.make_async_copy(w_hbm.at[pl.ds(0, _GR)], buf, sem).wait()

    for b in range(_NB):
        fire(b, bufs[b], sems[b])

    def accum(buf, c, acc):
        rbase = c * _GR

        def body16(q, a):
            new = list(a)
            s_vec = spkf_v[pl.ds(rbase + q * 16, 16)]
            for jj in range(16):
                s = lax.gather(
                    s_vec,
                    jnp.full((16, 1), jj, jnp.int32),
                    lax.GatherDimensionNumbers(
                        offset_dims=(),
                        collapsed_slice_dims=(0,),
                        start_index_map=(0,),
                    ),
                    (1,),
                    mode=lax.GatherScatterMode.PROMISE_IN_BOUNDS,
                )
                for v in range(_V8):
                    new[v] = new[v] + buf[q * 16 + jj, pl.ds(v * 16, 16)] * s
            return tuple(new)

        return lax.fori_loop(0, _GR // 16, body16, acc)

    init = tuple(jnp.zeros((16,), jnp.float32) for _ in range(_V8))

    def rev(i, accs):
        for b in range(_NB):
            c = i * _NB + b
            drain(bufs[b], sems[b])
            accs = accum(bufs[b], c, accs)
            cf = jnp.minimum(c + _NB, trips - 1)
            fire(cf, bufs[b], sems[b])
        return accs

    accs = lax.fori_loop(0, revs, rev, init)

    for b in range(_NB):
        drain(bufs[b], sems[b])

    for v in range(_V8):
        acc_v[0, pl.ds(v * 16, 16)] = accs[v]
    pltpu.sync_copy(acc_v, out_hbm.at[pl.ds(wid, 1)])


_sc_call = functools.partial(
    pl.kernel,
    out_type=jax.ShapeDtypeStruct((_NW, _NEU_OUT), jnp.float32),
    mesh=plsc.VectorSubcoreMesh(core_axis_name="c", subcore_axis_name="s"),
    compiler_params=pltpu.CompilerParams(needs_layout_passes=False),
    scratch_types=[
        pltpu.VMEM((_ROWS_W,), jnp.int32),
        pltpu.VMEM((_ROWS_W,), jnp.float32),
    ] + [pltpu.VMEM((_GR, _NEU_OUT), jnp.float32) for _ in range(_NB)] + [
        pltpu.VMEM((1, _NEU_OUT), jnp.float32),
    ] + [pltpu.SemaphoreType.DMA for _ in range(_NB)],
)(_sc_body)


def kernel(spikes_in, W, mempot):
    spikes_pad = (
        jnp.zeros((_PAD_IN,), jnp.int32).at[:_NEU_IN].set(spikes_in.astype(jnp.int32))
    )
    partials = _sc_call(spikes_pad, W)
    m = mempot + jnp.sum(partials, axis=0)
    spikes_out = m >= _THRES
    mnew = jnp.where(spikes_out, m - _THRES, (m * _DECAY - m) / _DECAY)
    traces_out = jnp.zeros((_NEU_OUT,), jnp.float32)
    return (spikes_out, traces_out, mnew)


# speedup vs baseline: 1.2441x; 1.2441x over previous
import functools

import jax
import jax.numpy as jnp
from jax import lax
from jax.experimental import pallas as pl
from jax.experimental.pallas import tpu as pltpu
from jax.experimental.pallas import tpu_sc as plsc

_NEU_IN = 100000
_NEU_OUT = 128
_THRES = 1.0
_DECAY = 2.0 ** 4

_NW = 32
_ROWS_W = 3200
_PAD_IN = _NW * _ROWS_W
_CHUNKS = _ROWS_W // 16
_GR = 160
_NB = 5
_V8 = _NEU_OUT // 16


def _sc_body(spikes_hbm, w_hbm, out_hbm, spk_v, spkf_v,
             b0, b1, b2, b3, b4, acc_v, s0, s1, s2, s3, s4):
    bufs = (b0, b1, b2, b3, b4)
    sems = (s0, s1, s2, s3, s4)

    wid = lax.axis_index("s") * 2 + lax.axis_index("c")
    base = wid * _ROWS_W
    nvalid = jnp.minimum(_ROWS_W, _NEU_IN - base)
    trips = lax.div(nvalid, _GR)
    revs = lax.div(trips, _NB)

    pltpu.sync_copy(spikes_hbm.at[pl.ds(base, _ROWS_W)], spk_v)

    def conv(c, x):
        spkf_v[pl.ds(c * 16, 16)] = spk_v[pl.ds(c * 16, 16)].astype(jnp.float32)
        return x

    lax.fori_loop(0, _CHUNKS, conv, 0)

    def fire(c, buf, sem):
        pltpu.async_copy(w_hbm.at[pl.ds(base + c * _GR, _GR)], buf, sem)

    def drain(buf, sem):
        pltpu.make_async_copy(w_hbm.at[pl.ds(0, _GR)], buf, sem).wait()

    for b in range(_NB):
        fire(b, bufs[b], sems[b])

    def accum(buf, c, acc):
        rbase = c * _GR

        def body4(q, a):
            new = list(a)
            for jj in range(4):
                ridx = jnp.full((16,), rbase + q * 4 + jj, jnp.int32)
                s = plsc.load_gather(spkf_v, [ridx])
                for v in range(_V8):
                    new[v] = new[v] + buf[q * 4 + jj, pl.ds(v * 16, 16)] * s
            return tuple(new)

        return lax.fori_loop(0, _GR // 4, body4, acc)

    init = tuple(jnp.zeros((16,), jnp.float32) for _ in range(_V8))

    def rev(i, accs):
        for b in range(_NB):
            c = i * _NB + b
            drain(bufs[b], sems[b])
            accs = accum(bufs[b], c, accs)
            cf = jnp.minimum(c + _NB, trips - 1)
            fire(cf, bufs[b], sems[b])
        return accs

    accs = lax.fori_loop(0, revs, rev, init)

    for b in range(_NB):
        drain(bufs[b], sems[b])

    for v in range(_V8):
        acc_v[0, pl.ds(v * 16, 16)] = accs[v]
    pltpu.sync_copy(acc_v, out_hbm.at[pl.ds(wid, 1)])


_sc_call = functools.partial(
    pl.kernel,
    out_type=jax.ShapeDtypeStruct((_NW, _NEU_OUT), jnp.float32),
    mesh=plsc.VectorSubcoreMesh(core_axis_name="c", subcore_axis_name="s"),
    compiler_params=pltpu.CompilerParams(needs_layout_passes=False),
    scratch_types=[
        pltpu.VMEM((_ROWS_W,), jnp.int32),
        pltpu.VMEM((_ROWS_W,), jnp.float32),
    ] + [pltpu.VMEM((_GR, _NEU_OUT), jnp.float32) for _ in range(_NB)] + [
        pltpu.VMEM((1, _NEU_OUT), jnp.float32),
    ] + [pltpu.SemaphoreType.DMA for _ in range(_NB)],
)(_sc_body)


def kernel(spikes_in, W, mempot):
    spikes_pad = (
        jnp.zeros((_PAD_IN,), jnp.int32).at[:_NEU_IN].set(spikes_in.astype(jnp.int32))
    )
    partials = _sc_call(spikes_pad, W)
    m = mempot + jnp.sum(partials, axis=0)
    spikes_out = m >= _THRES
    mnew = jnp.where(spikes_out, m - _THRES, (m * _DECAY - m) / _DECAY)
    traces_out = jnp.zeros((_NEU_OUT,), jnp.float32)
    return (spikes_out, traces_out, mnew)
